# Initial kernel scaffold; baseline (speedup 1.0000x reference)
import jax
import jax.numpy as jnp
from jax.experimental import pallas as pl


def kernel(x, node_emb, rel_emb, W, W_loop, W_rel, edge_index, edge_type):
    raise NotImplementedError("write your pallas kernel here")


# speedup vs baseline: 1.8230x; 1.8230x over previous
import dataclasses
import functools

import jax
import jax.numpy as jnp
from jax import lax
from jax.experimental import pallas as pl
from jax.experimental.pallas import tpu as pltpu
from jax.experimental.pallas import tpu_sc as plsc

N = 10000
E = 320000
D = 128
R = 50
RP = 64
NC = 2
NS = 16
NW = NC * NS
EW = E // NW
K = 80
NCH = EW // K
ES = E // NS
NCHE = ES // K
DH = D // NC
RH = RP // NC
SROWS = 624
ZR = 208
REM = N - NS * SROWS

_mesh = plsc.VectorSubcoreMesh(core_axis_name="c", subcore_axis_name="s")

_cp = pltpu.CompilerParams()
for _f, _v in (("needs_layout_passes", False), ("use_tc_tiling_on_sc", False)):
    if _f in pltpu.CompilerParams.__dataclass_fields__:
        _cp = dataclasses.replace(_cp, **{_f: _v})


def _worker_id():
    return lax.axis_index("c") * NS + lax.axis_index("s")


@functools.partial(
    pl.kernel,
    out_type=jax.ShapeDtypeStruct((NC, N, DH), jnp.float32),
    mesh=_mesh,
    scratch_types=[
        pltpu.VMEM((NCHE, K), jnp.int32),
        pltpu.VMEM((NCHE, K), jnp.int32),
        pltpu.VMEM((K,), jnp.int32),
        pltpu.VMEM((K, DH), jnp.float32),
        pltpu.VMEM((ZR, DH), jnp.float32),
        pltpu.VMEM_SHARED((N, DH), jnp.float32),
    ],
    compiler_params=_cp,
)
def _encode(src_hbm, dst_hbm, emb2_hbm, aggp_hbm,
            src_v, dst_v, gidx_v, rows_v, za_v, agg_sh):
    c = lax.axis_index("c")
    s = lax.axis_index("s")
    zf = jnp.zeros((16,), jnp.float32)

    @pl.loop(0, ZR)
    def _(i):
        for j in range(DH // 16):
            za_v[i, pl.ds(j * 16, 16)] = zf

    @pl.loop(0, SROWS // ZR)
    def _(j):
        row0 = s * SROWS + j * ZR
        pltpu.sync_copy(za_v, agg_sh.at[pl.ds(row0, ZR)])

    @pl.when(s == NS - 1)
    def _():
        pltpu.sync_copy(za_v.at[pl.ds(0, REM)], agg_sh.at[pl.ds(NS * SROWS, REM)])

    pltpu.sync_copy(src_hbm.at[s], src_v)
    pltpu.sync_copy(dst_hbm.at[s], dst_v)

    plsc.subcore_barrier()

    @pl.loop(0, NCHE)
    def _(i):
        for g in range(K // 16):
            s16 = src_v[i, pl.ds(g * 16, 16)]
            gidx_v[pl.ds(g * 16, 16)] = s16 * 2 + c
        pltpu.sync_copy(emb2_hbm.at[gidx_v], rows_v)
        pltpu.sync_copy(rows_v, agg_sh.at[dst_v.at[i]], add=True)

    plsc.subcore_barrier()

    @pl.loop(0, SROWS // ZR)
    def _(j):
        row0 = s * SROWS + j * ZR
        pltpu.sync_copy(agg_sh.at[pl.ds(row0, ZR)], aggp_hbm.at[c, pl.ds(row0, ZR)])

    @pl.when(s == NS - 1)
    def _():
        pltpu.sync_copy(agg_sh.at[pl.ds(NS * SROWS, REM)],
                        aggp_hbm.at[c, pl.ds(NS * SROWS, REM)])


@functools.partial(
    pl.kernel,
    out_type=jax.ShapeDtypeStruct((NC, N, RH), jnp.float32),
    mesh=_mesh,
    scratch_types=[
        pltpu.VMEM((NCHE, K), jnp.int32),
        pltpu.VMEM((NCHE, K), jnp.int32),
        pltpu.VMEM((K, RH), jnp.float32),
        pltpu.VMEM((ZR, RH), jnp.float32),
        pltpu.VMEM_SHARED((N, RH), jnp.float32),
    ],
    compiler_params=_cp,
)
def _counts(dst_hbm, et_hbm, cntp_hbm, dst_v, et_v, oh_v, zc_v, cnt_sh):
    c = lax.axis_index("c")
    s = lax.axis_index("s")
    zf = jnp.zeros((16,), jnp.float32)
    ones = jnp.ones((16,), jnp.float32)
    iota = lax.iota(jnp.int32, 16)
    rlo = c * RH

    @pl.loop(0, ZR)
    def _(i):
        for j in range(RH // 16):
            zc_v[i, pl.ds(j * 16, 16)] = zf

    @pl.loop(0, K)
    def _(i):
        for j in range(RH // 16):
            oh_v[i, pl.ds(j * 16, 16)] = zf

    @pl.loop(0, SROWS // ZR)
    def _(j):
        row0 = s * SROWS + j * ZR
        pltpu.sync_copy(zc_v, cnt_sh.at[pl.ds(row0, ZR)])

    @pl.when(s == NS - 1)
    def _():
        pltpu.sync_copy(zc_v.at[pl.ds(0, REM)], cnt_sh.at[pl.ds(NS * SROWS, REM)])

    pltpu.sync_copy(dst_hbm.at[s], dst_v)
    pltpu.sync_copy(et_hbm.at[s], et_v)

    plsc.subcore_barrier()

    @pl.loop(0, NCHE)
    def _(i):
        for g in range(K // 16):
            row16 = g * 16 + iota
            et16 = et_v[i, pl.ds(g * 16, 16)]
            etl16 = et16 - rlo
            m16 = (et16 >= rlo) & (etl16 < RH)
            plsc.store_scatter(oh_v, [row16, etl16], ones, mask=m16)
        pltpu.sync_copy(oh_v, cnt_sh.at[dst_v.at[i]], add=True)
        for g in range(K // 16):
            row16 = g * 16 + iota
            et16 = et_v[i, pl.ds(g * 16, 16)]
            etl16 = et16 - rlo
            m16 = (et16 >= rlo) & (etl16 < RH)
            plsc.store_scatter(oh_v, [row16, etl16], zf, mask=m16)

    plsc.subcore_barrier()

    @pl.loop(0, SROWS // ZR)
    def _(j):
        row0 = s * SROWS + j * ZR
        pltpu.sync_copy(cnt_sh.at[pl.ds(row0, ZR)], cntp_hbm.at[c, pl.ds(row0, ZR)])

    @pl.when(s == NS - 1)
    def _():
        pltpu.sync_copy(cnt_sh.at[pl.ds(NS * SROWS, REM)],
                        cntp_hbm.at[c, pl.ds(NS * SROWS, REM)])


BN = 1000


def _dense_body(aggp_ref, cntp_ref, emb_ref, relh_ref, w_ref, wl_ref, wr_ref,
                h_ref, r_ref):
    deg = (jnp.sum(cntp_ref[0], axis=1, keepdims=True)
           + jnp.sum(cntp_ref[1], axis=1, keepdims=True))
    agg = jnp.concatenate([aggp_ref[0], aggp_ref[1]], axis=1)
    agg = agg - (jnp.dot(cntp_ref[0], relh_ref[0],
                         preferred_element_type=jnp.float32)
                 + jnp.dot(cntp_ref[1], relh_ref[1],
                           preferred_element_type=jnp.float32))
    agg = agg / jnp.maximum(deg, 1.0)
    h = jnp.dot(agg, w_ref[...], preferred_element_type=jnp.float32)
    h = h + jnp.dot(emb_ref[...], wl_ref[...], preferred_element_type=jnp.float32)
    h_ref[...] = jnp.tanh(h)
    relp = jnp.concatenate([relh_ref[0], relh_ref[1]], axis=0)
    r_ref[...] = jnp.dot(relp, wr_ref[...], preferred_element_type=jnp.float32)


_dense = pl.pallas_call(
    _dense_body,
    grid=(N // BN,),
    in_specs=[
        pl.BlockSpec((NC, BN, DH), lambda i: (0, i, 0)),
        pl.BlockSpec((NC, BN, RH), lambda i: (0, i, 0)),
        pl.BlockSpec((BN, D), lambda i: (i, 0)),
        pl.BlockSpec((NC, RH, D), lambda i: (0, 0, 0)),
        pl.BlockSpec((D, D), lambda i: (0, 0)),
        pl.BlockSpec((D, D), lambda i: (0, 0)),
        pl.BlockSpec((D, D), lambda i: (0, 0)),
    ],
    out_specs=[
        pl.BlockSpec((BN, D), lambda i: (i, 0)),
        pl.BlockSpec((RP, D), lambda i: (0, 0)),
    ],
    out_shape=[
        jax.ShapeDtypeStruct((N, D), jnp.float32),
        jax.ShapeDtypeStruct((RP, D), jnp.float32),
    ],
)


@functools.partial(
    pl.kernel,
    out_type=jax.ShapeDtypeStruct((E,), jnp.float32),
    mesh=_mesh,
    scratch_types=[
        pltpu.VMEM((NCH, K), jnp.int32),
        pltpu.VMEM((NCH, K), jnp.int32),
        pltpu.VMEM((NCH, K), jnp.int32),
        pltpu.VMEM((K, D), jnp.float32),
        pltpu.VMEM((K, D), jnp.float32),
        pltpu.VMEM((RP, D), jnp.float32),
        pltpu.VMEM((K,), jnp.float32),
    ],
    compiler_params=_cp,
)
def _decode(h_hbm, r_hbm, src_hbm, dst_hbm, et_hbm, out_hbm,
            src_v, dst_v, et_v, a_v, b_v, rt_v, out_v):
    wid = _worker_id()
    iota = lax.iota(jnp.int32, 16)

    pltpu.sync_copy(r_hbm, rt_v)
    pltpu.sync_copy(src_hbm.at[wid], src_v)
    pltpu.sync_copy(dst_hbm.at[wid], dst_v)
    pltpu.sync_copy(et_hbm.at[wid], et_v)

    @pl.loop(0, NCH)
    def _(i):
        pltpu.sync_copy(h_hbm.at[src_v.at[i]], a_v)
        pltpu.sync_copy(h_hbm.at[dst_v.at[i]], b_v)
        for g in range(K // 16):
            row16 = g * 16 + iota
            et16 = et_v[i, pl.ds(g * 16, 16)]

            def body(d4, accs):
                a0, a1, a2, a3 = accs
                new = []
                for k, acc in zip(range(4), (a0, a1, a2, a3)):
                    col = jnp.full((16,), 0, jnp.int32) + (d4 * 4 + k)
                    a = plsc.load_gather(a_v, [row16, col])
                    b = plsc.load_gather(b_v, [row16, col])
                    rr = plsc.load_gather(rt_v, [et16, col])
                    new.append(acc + a * b * rr)
                return tuple(new)

            z = jnp.zeros((16,), jnp.float32)
            a0, a1, a2, a3 = lax.fori_loop(0, D // 4, body, (z, z, z, z))
            out_v[pl.ds(g * 16, 16)] = (a0 + a1) + (a2 + a3)
        pltpu.sync_copy(out_v, out_hbm.at[pl.ds(wid * EW + i * K, K)])


@jax.jit
def kernel(x, node_emb, rel_emb, W, W_loop, W_rel, edge_index, edge_type):
    del x
    src = edge_index[0]
    dst = edge_index[1]
    et = edge_type
    emb2 = node_emb.reshape(N * NC, DH)
    relh = jnp.pad(rel_emb, ((0, RP - R), (0, 0))).reshape(NC, RH, D)
    aggp = _encode(src.reshape(NS, NCHE, K), dst.reshape(NS, NCHE, K), emb2)
    cntp = _counts(dst.reshape(NS, NCHE, K), et.reshape(NS, NCHE, K))
    h, r = _dense(aggp, cntp, node_emb, relh, W, W_loop, W_rel)
    return _decode(h, r, src.reshape(NW, NCH, K), dst.reshape(NW, NCH, K),
                   et.reshape(NW, NCH, K))


# speedup vs baseline: 2.1202x; 1.1630x over previous
import dataclasses
import functools

import jax
import jax.numpy as jnp
from jax import lax
from jax.experimental import pallas as pl
from jax.experimental.pallas import tpu as pltpu
from jax.experimental.pallas import tpu_sc as plsc

N = 10000
E = 320000
D = 128
R = 50
RP = 64
NC = 2
NS = 16
NW = NC * NS
EW = E // NW
K = 80
NCH = EW // K
ES = E // NS
NCHE = ES // K
DH = D // NC
RH = RP // NC
SROWS = 624
ZR = 208
REM = N - NS * SROWS

_mesh = plsc.VectorSubcoreMesh(core_axis_name="c", subcore_axis_name="s")

_cp = pltpu.CompilerParams()
for _f, _v in (("needs_layout_passes", False), ("use_tc_tiling_on_sc", False)):
    if _f in pltpu.CompilerParams.__dataclass_fields__:
        _cp = dataclasses.replace(_cp, **{_f: _v})


def _worker_id():
    return lax.axis_index("c") * NS + lax.axis_index("s")


@functools.partial(
    pl.kernel,
    out_type=jax.ShapeDtypeStruct((NC, N, DH), jnp.float32),
    mesh=_mesh,
    scratch_types=[
        pltpu.VMEM((NCHE, K), jnp.int32),
        pltpu.VMEM((NCHE, K), jnp.int32),
        pltpu.VMEM((4, K), jnp.int32),
        pltpu.VMEM((4, K, DH), jnp.float32),
        pltpu.VMEM((ZR, DH), jnp.float32),
        pltpu.VMEM_SHARED((N, DH), jnp.float32),
        pltpu.SemaphoreType.DMA,
        pltpu.SemaphoreType.DMA,
        pltpu.SemaphoreType.DMA,
        pltpu.SemaphoreType.DMA,
        pltpu.SemaphoreType.DMA,
        pltpu.SemaphoreType.DMA,
        pltpu.SemaphoreType.DMA,
        pltpu.SemaphoreType.DMA,
    ],
    compiler_params=_cp,
)
def _encode(src_hbm, dst_hbm, emb2_hbm, aggp_hbm,
            src_v, dst_v, gidx_v, rows_v, za_v, agg_sh,
            sg0, sg1, sg2, sg3, ss0, ss1, ss2, ss3):
    c = lax.axis_index("c")
    s = lax.axis_index("s")
    zf = jnp.zeros((16,), jnp.float32)
    sg = (sg0, sg1, sg2, sg3)
    ss = (ss0, ss1, ss2, ss3)

    @pl.loop(0, ZR)
    def _(i):
        for j in range(DH // 16):
            za_v[i, pl.ds(j * 16, 16)] = zf

    @pl.loop(0, SROWS // ZR)
    def _(j):
        row0 = s * SROWS + j * ZR
        pltpu.sync_copy(za_v, agg_sh.at[pl.ds(row0, ZR)])

    @pl.when(s == NS - 1)
    def _():
        pltpu.sync_copy(za_v.at[pl.ds(0, REM)], agg_sh.at[pl.ds(NS * SROWS, REM)])

    pltpu.sync_copy(src_hbm.at[s], src_v)
    pltpu.sync_copy(dst_hbm.at[s], dst_v)

    plsc.subcore_barrier()

    def start_gather(j, p):
        for g in range(K // 16):
            s16 = src_v[j, pl.ds(g * 16, 16)]
            gidx_v[p, pl.ds(g * 16, 16)] = s16 * 2 + c
        pltpu.async_copy(emb2_hbm.at[gidx_v.at[p]], rows_v.at[p], sg[p])

    def wait_gather(j, p):
        pltpu.make_async_copy(emb2_hbm.at[gidx_v.at[p]], rows_v.at[p],
                              sg[p]).wait()

    def start_scatter(j, p):
        pltpu.async_copy(rows_v.at[p], agg_sh.at[dst_v.at[j]], ss[p], add=True)

    def wait_scatter(j, p):
        pltpu.make_async_copy(rows_v.at[p], agg_sh.at[dst_v.at[j]],
                              ss[p]).wait()

    start_gather(0, 0)
    start_gather(1, 1)

    @pl.loop(0, NCHE - 2, step=4)
    def _(i):
        for p in range(4):
            j = i + p
            wait_gather(j, p)
            start_scatter(j, p)
            q = (p + 2) % 4

            @pl.when(j >= 2)
            def _():
                wait_scatter(j - 2, q)

            start_gather(j + 2, q)

    for j, p in ((NCHE - 2, (NCHE - 2) % 4), (NCHE - 1, (NCHE - 1) % 4)):
        wait_gather(j, p)
        start_scatter(j, p)
    for j in range(NCHE - 4, NCHE):
        wait_scatter(j, j % 4)

    plsc.subcore_barrier()

    @pl.loop(0, SROWS // ZR)
    def _(j):
        row0 = s * SROWS + j * ZR
        pltpu.sync_copy(agg_sh.at[pl.ds(row0, ZR)], aggp_hbm.at[c, pl.ds(row0, ZR)])

    @pl.when(s == NS - 1)
    def _():
        pltpu.sync_copy(agg_sh.at[pl.ds(NS * SROWS, REM)],
                        aggp_hbm.at[c, pl.ds(NS * SROWS, REM)])


@functools.partial(
    pl.kernel,
    out_type=jax.ShapeDtypeStruct((NC, N, RH), jnp.float32),
    mesh=_mesh,
    scratch_types=[
        pltpu.VMEM((NCHE, K), jnp.int32),
        pltpu.VMEM((NCHE, K), jnp.int32),
        pltpu.VMEM((K, RH), jnp.float32),
        pltpu.VMEM((ZR, RH), jnp.float32),
        pltpu.VMEM_SHARED((N, RH), jnp.float32),
    ],
    compiler_params=_cp,
)
def _counts(dst_hbm, et_hbm, cntp_hbm, dst_v, et_v, oh_v, zc_v, cnt_sh):
    c = lax.axis_index("c")
    s = lax.axis_index("s")
    zf = jnp.zeros((16,), jnp.float32)
    ones = jnp.ones((16,), jnp.float32)
    iota = lax.iota(jnp.int32, 16)
    rlo = c * RH

    @pl.loop(0, ZR)
    def _(i):
        for j in range(RH // 16):
            zc_v[i, pl.ds(j * 16, 16)] = zf

    @pl.loop(0, K)
    def _(i):
        for j in range(RH // 16):
            oh_v[i, pl.ds(j * 16, 16)] = zf

    @pl.loop(0, SROWS // ZR)
    def _(j):
        row0 = s * SROWS + j * ZR
        pltpu.sync_copy(zc_v, cnt_sh.at[pl.ds(row0, ZR)])

    @pl.when(s == NS - 1)
    def _():
        pltpu.sync_copy(zc_v.at[pl.ds(0, REM)], cnt_sh.at[pl.ds(NS * SROWS, REM)])

    pltpu.sync_copy(dst_hbm.at[s], dst_v)
    pltpu.sync_copy(et_hbm.at[s], et_v)

    plsc.subcore_barrier()

    @pl.loop(0, NCHE)
    def _(i):
        for g in range(K // 16):
            row16 = g * 16 + iota
            et16 = et_v[i, pl.ds(g * 16, 16)]
            etl16 = et16 - rlo
            m16 = (et16 >= rlo) & (etl16 < RH)
            plsc.store_scatter(oh_v, [row16, etl16], ones, mask=m16)
        pltpu.sync_copy(oh_v, cnt_sh.at[dst_v.at[i]], add=True)
        for g in range(K // 16):
            row16 = g * 16 + iota
            et16 = et_v[i, pl.ds(g * 16, 16)]
            etl16 = et16 - rlo
            m16 = (et16 >= rlo) & (etl16 < RH)
            plsc.store_scatter(oh_v, [row16, etl16], zf, mask=m16)

    plsc.subcore_barrier()

    @pl.loop(0, SROWS // ZR)
    def _(j):
        row0 = s * SROWS + j * ZR
        pltpu.sync_copy(cnt_sh.at[pl.ds(row0, ZR)], cntp_hbm.at[c, pl.ds(row0, ZR)])

    @pl.when(s == NS - 1)
    def _():
        pltpu.sync_copy(cnt_sh.at[pl.ds(NS * SROWS, REM)],
                        cntp_hbm.at[c, pl.ds(NS * SROWS, REM)])


BN = 1000


def _dense_body(aggp_ref, cntp_ref, emb_ref, relh_ref, w_ref, wl_ref, wr_ref,
                h_ref, r_ref):
    deg = (jnp.sum(cntp_ref[0], axis=1, keepdims=True)
           + jnp.sum(cntp_ref[1], axis=1, keepdims=True))
    agg = jnp.concatenate([aggp_ref[0], aggp_ref[1]], axis=1)
    agg = agg - (jnp.dot(cntp_ref[0], relh_ref[0],
                         preferred_element_type=jnp.float32)
                 + jnp.dot(cntp_ref[1], relh_ref[1],
                           preferred_element_type=jnp.float32))
    agg = agg / jnp.maximum(deg, 1.0)
    h = jnp.dot(agg, w_ref[...], preferred_element_type=jnp.float32)
    h = h + jnp.dot(emb_ref[...], wl_ref[...], preferred_element_type=jnp.float32)
    h_ref[...] = jnp.tanh(h)
    relp = jnp.concatenate([relh_ref[0], relh_ref[1]], axis=0)
    r_ref[...] = jnp.dot(relp, wr_ref[...], preferred_element_type=jnp.float32)


_dense = pl.pallas_call(
    _dense_body,
    grid=(N // BN,),
    in_specs=[
        pl.BlockSpec((NC, BN, DH), lambda i: (0, i, 0)),
        pl.BlockSpec((NC, BN, RH), lambda i: (0, i, 0)),
        pl.BlockSpec((BN, D), lambda i: (i, 0)),
        pl.BlockSpec((NC, RH, D), lambda i: (0, 0, 0)),
        pl.BlockSpec((D, D), lambda i: (0, 0)),
        pl.BlockSpec((D, D), lambda i: (0, 0)),
        pl.BlockSpec((D, D), lambda i: (0, 0)),
    ],
    out_specs=[
        pl.BlockSpec((BN, D), lambda i: (i, 0)),
        pl.BlockSpec((RP, D), lambda i: (0, 0)),
    ],
    out_shape=[
        jax.ShapeDtypeStruct((N, D), jnp.float32),
        jax.ShapeDtypeStruct((RP, D), jnp.float32),
    ],
)


@functools.partial(
    pl.kernel,
    out_type=jax.ShapeDtypeStruct((NW, NCH, K), jnp.float32),
    mesh=_mesh,
    scratch_types=[
        pltpu.VMEM((NCH, K), jnp.int32),
        pltpu.VMEM((NCH, K), jnp.int32),
        pltpu.VMEM((NCH, K), jnp.int32),
        pltpu.VMEM((K, D), jnp.float32),
        pltpu.VMEM((K, D), jnp.float32),
        pltpu.VMEM((K, D), jnp.float32),
        pltpu.VMEM((K, D), jnp.float32),
        pltpu.VMEM((RP * D,), jnp.float32),
        pltpu.VMEM((NCH, K), jnp.float32),
        pltpu.SemaphoreType.DMA,
        pltpu.SemaphoreType.DMA,
    ],
    compiler_params=_cp,
)
def _decode(h_hbm, r_hbm, src_hbm, dst_hbm, et_hbm, out_hbm,
            src_v, dst_v, et_v, a0_v, a1_v, b0_v, b1_v, rt_v, out_v, g0, g1):
    wid = _worker_id()
    iota = lax.iota(jnp.int32, 16)
    zf = jnp.zeros((16,), jnp.float32)

    pltpu.sync_copy(r_hbm, rt_v)
    pltpu.sync_copy(src_hbm.at[wid], src_v)
    pltpu.sync_copy(dst_hbm.at[wid], dst_v)
    pltpu.sync_copy(et_hbm.at[wid], et_v)

    bufs = ((a0_v, b0_v, g0), (a1_v, b1_v, g1))

    def start_gather(j, p):
        a_v, b_v, sem = bufs[p]
        pltpu.async_copy(h_hbm.at[src_v.at[j]], a_v, sem)
        pltpu.async_copy(h_hbm.at[dst_v.at[j]], b_v, sem)

    def wait_gather(j, p):
        a_v, b_v, sem = bufs[p]
        pltpu.make_async_copy(h_hbm.at[src_v.at[j]], a_v, sem).wait()
        pltpu.make_async_copy(h_hbm.at[dst_v.at[j]], b_v, sem).wait()

    def compute(j, p):
        a_v, b_v, _ = bufs[p]
        for g in range(K // 16):
            row16 = g * 16 + iota
            et16 = et_v[j, pl.ds(g * 16, 16)]
            rbase = et16 * D

            def body(k, accs):
                d0 = k * 8
                new = list(accs)
                for t in range(8):
                    col = jnp.full((16,), 0, jnp.int32) + (d0 + t)
                    a = plsc.load_gather(a_v, [row16, col])
                    b = plsc.load_gather(b_v, [row16, col])
                    rr = plsc.load_gather(rt_v, [rbase + col])
                    new[t % 4] = new[t % 4] + a * b * rr
                return tuple(new)

            s0, s1, s2, s3 = lax.fori_loop(0, D // 8, body, (zf, zf, zf, zf))
            out_v[j, pl.ds(g * 16, 16)] = (s0 + s1) + (s2 + s3)

    start_gather(0, 0)
    start_gather(1, 1)

    @pl.loop(0, NCH - 1, step=2)
    def _(i):
        wait_gather(i, 0)
        compute(i, 0)
        start_gather(i + 2, 0)
        wait_gather(i + 1, 1)
        compute(i + 1, 1)

        @pl.when(i + 3 < NCH)
        def _():
            start_gather(i + 3, 1)

    wait_gather(NCH - 1, 0)
    compute(NCH - 1, 0)
    pltpu.sync_copy(out_v, out_hbm.at[wid])


@jax.jit
def kernel(x, node_emb, rel_emb, W, W_loop, W_rel, edge_index, edge_type):
    del x
    src = edge_index[0]
    dst = edge_index[1]
    et = edge_type
    emb2 = node_emb.reshape(N * NC, DH)
    relh = jnp.pad(rel_emb, ((0, RP - R), (0, 0))).reshape(NC, RH, D)
    aggp = _encode(src.reshape(NS, NCHE, K), dst.reshape(NS, NCHE, K), emb2)
    cntp = _counts(dst.reshape(NS, NCHE, K), et.reshape(NS, NCHE, K))
    h, r = _dense(aggp, cntp, node_emb, relh, W, W_loop, W_rel)
    scores = _decode(h, r.reshape(RP * D), src.reshape(NW, NCH, K),
                     dst.reshape(NW, NCH, K), et.reshape(NW, NCH, K))
    return scores.reshape(E)


# speedup vs baseline: 6.5975x; 3.1117x over previous
import dataclasses
import functools

import jax
import jax.numpy as jnp
from jax import lax
from jax.experimental import pallas as pl
from jax.experimental.pallas import tpu as pltpu
from jax.experimental.pallas import tpu_sc as plsc

N = 10000
E = 320000
D = 128
R = 50
RP = 64
NC = 2
NS = 16
NW = NC * NS
EW = E // NW
K = 80
NCH = EW // K
ES = E // NS
NCHE = ES // K
DH = D // NC
RH = RP // NC
SROWS = 624
ZR = 208
REM = N - NS * SROWS

_mesh = plsc.VectorSubcoreMesh(core_axis_name="c", subcore_axis_name="s")

_cp = pltpu.CompilerParams()
for _f, _v in (("needs_layout_passes", False), ("use_tc_tiling_on_sc", False)):
    if _f in pltpu.CompilerParams.__dataclass_fields__:
        _cp = dataclasses.replace(_cp, **{_f: _v})


def _worker_id():
    return lax.axis_index("c") * NS + lax.axis_index("s")


@functools.partial(
    pl.kernel,
    out_type=jax.ShapeDtypeStruct((NC, N, DH), jnp.float32),
    mesh=_mesh,
    scratch_types=[
        pltpu.VMEM((NCHE, K), jnp.int32),
        pltpu.VMEM((NCHE, K), jnp.int32),
        pltpu.VMEM((4, K), jnp.int32),
        pltpu.VMEM((4, K, DH), jnp.float32),
        pltpu.VMEM((ZR, DH), jnp.float32),
        pltpu.VMEM_SHARED((N, DH), jnp.float32),
        pltpu.SemaphoreType.DMA,
        pltpu.SemaphoreType.DMA,
        pltpu.SemaphoreType.DMA,
        pltpu.SemaphoreType.DMA,
        pltpu.SemaphoreType.DMA,
        pltpu.SemaphoreType.DMA,
        pltpu.SemaphoreType.DMA,
        pltpu.SemaphoreType.DMA,
    ],
    compiler_params=_cp,
)
def _encode(src_hbm, dst_hbm, emb2_hbm, aggp_hbm,
            src_v, dst_v, gidx_v, rows_v, za_v, agg_sh,
            sg0, sg1, sg2, sg3, ss0, ss1, ss2, ss3):
    c = lax.axis_index("c")
    s = lax.axis_index("s")
    zf = jnp.zeros((16,), jnp.float32)
    sg = (sg0, sg1, sg2, sg3)
    ss = (ss0, ss1, ss2, ss3)

    @pl.loop(0, ZR)
    def _(i):
        for j in range(DH // 16):
            za_v[i, pl.ds(j * 16, 16)] = zf

    @pl.loop(0, SROWS // ZR)
    def _(j):
        row0 = s * SROWS + j * ZR
        pltpu.sync_copy(za_v, agg_sh.at[pl.ds(row0, ZR)])

    @pl.when(s == NS - 1)
    def _():
        pltpu.sync_copy(za_v.at[pl.ds(0, REM)], agg_sh.at[pl.ds(NS * SROWS, REM)])

    pltpu.sync_copy(src_hbm.at[s], src_v)
    pltpu.sync_copy(dst_hbm.at[s], dst_v)

    plsc.subcore_barrier()

    def start_gather(j, p):
        for g in range(K // 16):
            s16 = src_v[j, pl.ds(g * 16, 16)]
            gidx_v[p, pl.ds(g * 16, 16)] = s16 * 2 + c
        pltpu.async_copy(emb2_hbm.at[gidx_v.at[p]], rows_v.at[p], sg[p])

    def wait_gather(j, p):
        pltpu.make_async_copy(emb2_hbm.at[gidx_v.at[p]], rows_v.at[p],
                              sg[p]).wait()

    def start_scatter(j, p):
        pltpu.async_copy(rows_v.at[p], agg_sh.at[dst_v.at[j]], ss[p], add=True)

    def wait_scatter(j, p):
        pltpu.make_async_copy(rows_v.at[p], agg_sh.at[dst_v.at[j]],
                              ss[p]).wait()

    start_gather(0, 0)
    start_gather(1, 1)

    @pl.loop(0, NCHE - 2, step=4)
    def _(i):
        for p in range(4):
            j = i + p
            wait_gather(j, p)
            start_scatter(j, p)
            q = (p + 2) % 4

            @pl.when(j >= 2)
            def _():
                wait_scatter(j - 2, q)

            start_gather(j + 2, q)

    for j, p in ((NCHE - 2, (NCHE - 2) % 4), (NCHE - 1, (NCHE - 1) % 4)):
        wait_gather(j, p)
        start_scatter(j, p)
    for j in range(NCHE - 4, NCHE):
        wait_scatter(j, j % 4)

    plsc.subcore_barrier()

    @pl.loop(0, SROWS // ZR)
    def _(j):
        row0 = s * SROWS + j * ZR
        pltpu.sync_copy(agg_sh.at[pl.ds(row0, ZR)], aggp_hbm.at[c, pl.ds(row0, ZR)])

    @pl.when(s == NS - 1)
    def _():
        pltpu.sync_copy(agg_sh.at[pl.ds(NS * SROWS, REM)],
                        aggp_hbm.at[c, pl.ds(NS * SROWS, REM)])


@functools.partial(
    pl.kernel,
    out_type=jax.ShapeDtypeStruct((NC, N, RH), jnp.float32),
    mesh=_mesh,
    scratch_types=[
        pltpu.VMEM((NCHE, K), jnp.int32),
        pltpu.VMEM((NCHE, K), jnp.int32),
        pltpu.VMEM((K, RH), jnp.float32),
        pltpu.VMEM((ZR, RH), jnp.float32),
        pltpu.VMEM_SHARED((N, RH), jnp.float32),
    ],
    compiler_params=_cp,
)
def _counts(dst_hbm, et_hbm, cntp_hbm, dst_v, et_v, oh_v, zc_v, cnt_sh):
    c = lax.axis_index("c")
    s = lax.axis_index("s")
    zf = jnp.zeros((16,), jnp.float32)
    ones = jnp.ones((16,), jnp.float32)
    iota = lax.iota(jnp.int32, 16)
    rlo = c * RH

    @pl.loop(0, ZR)
    def _(i):
        for j in range(RH // 16):
            zc_v[i, pl.ds(j * 16, 16)] = zf

    @pl.loop(0, K)
    def _(i):
        for j in range(RH // 16):
            oh_v[i, pl.ds(j * 16, 16)] = zf

    @pl.loop(0, SROWS // ZR)
    def _(j):
        row0 = s * SROWS + j * ZR
        pltpu.sync_copy(zc_v, cnt_sh.at[pl.ds(row0, ZR)])

    @pl.when(s == NS - 1)
    def _():
        pltpu.sync_copy(zc_v.at[pl.ds(0, REM)], cnt_sh.at[pl.ds(NS * SROWS, REM)])

    pltpu.sync_copy(dst_hbm.at[s], dst_v)
    pltpu.sync_copy(et_hbm.at[s], et_v)

    plsc.subcore_barrier()

    @pl.loop(0, NCHE)
    def _(i):
        for g in range(K // 16):
            row16 = g * 16 + iota
            et16 = et_v[i, pl.ds(g * 16, 16)]
            etl16 = et16 - rlo
            m16 = (et16 >= rlo) & (etl16 < RH)
            plsc.store_scatter(oh_v, [row16, etl16], ones, mask=m16)
        pltpu.sync_copy(oh_v, cnt_sh.at[dst_v.at[i]], add=True)
        for g in range(K // 16):
            row16 = g * 16 + iota
            et16 = et_v[i, pl.ds(g * 16, 16)]
            etl16 = et16 - rlo
            m16 = (et16 >= rlo) & (etl16 < RH)
            plsc.store_scatter(oh_v, [row16, etl16], zf, mask=m16)

    plsc.subcore_barrier()

    @pl.loop(0, SROWS // ZR)
    def _(j):
        row0 = s * SROWS + j * ZR
        pltpu.sync_copy(cnt_sh.at[pl.ds(row0, ZR)], cntp_hbm.at[c, pl.ds(row0, ZR)])

    @pl.when(s == NS - 1)
    def _():
        pltpu.sync_copy(cnt_sh.at[pl.ds(NS * SROWS, REM)],
                        cntp_hbm.at[c, pl.ds(NS * SROWS, REM)])


BN = 1000


def _dense_body(aggp_ref, cntp_ref, emb_ref, relh_ref, w_ref, wl_ref, wr_ref,
                h_ref, r_ref):
    deg = (jnp.sum(cntp_ref[0], axis=1, keepdims=True)
           + jnp.sum(cntp_ref[1], axis=1, keepdims=True))
    agg = jnp.concatenate([aggp_ref[0], aggp_ref[1]], axis=1)
    agg = agg - (jnp.dot(cntp_ref[0], relh_ref[0],
                         preferred_element_type=jnp.float32)
                 + jnp.dot(cntp_ref[1], relh_ref[1],
                           preferred_element_type=jnp.float32))
    agg = agg / jnp.maximum(deg, 1.0)
    h = jnp.dot(agg, w_ref[...], preferred_element_type=jnp.float32)
    h = h + jnp.dot(emb_ref[...], wl_ref[...], preferred_element_type=jnp.float32)
    h_ref[...] = jnp.tanh(h)
    relp = jnp.concatenate([relh_ref[0], relh_ref[1]], axis=0)
    r_ref[...] = jnp.dot(relp, wr_ref[...], preferred_element_type=jnp.float32)


_dense = pl.pallas_call(
    _dense_body,
    grid=(N // BN,),
    in_specs=[
        pl.BlockSpec((NC, BN, DH), lambda i: (0, i, 0)),
        pl.BlockSpec((NC, BN, RH), lambda i: (0, i, 0)),
        pl.BlockSpec((BN, D), lambda i: (i, 0)),
        pl.BlockSpec((NC, RH, D), lambda i: (0, 0, 0)),
        pl.BlockSpec((D, D), lambda i: (0, 0)),
        pl.BlockSpec((D, D), lambda i: (0, 0)),
        pl.BlockSpec((D, D), lambda i: (0, 0)),
    ],
    out_specs=[
        pl.BlockSpec((BN, D), lambda i: (i, 0)),
        pl.BlockSpec((RP, D), lambda i: (0, 0)),
    ],
    out_shape=[
        jax.ShapeDtypeStruct((N, D), jnp.float32),
        jax.ShapeDtypeStruct((RP, D), jnp.float32),
    ],
)


@functools.partial(
    pl.kernel,
    out_type=jax.ShapeDtypeStruct((NW, NCH, K, D), jnp.float32),
    mesh=_mesh,
    scratch_types=[
        pltpu.VMEM((NCH, K), jnp.int32),
        pltpu.VMEM((NCH, K), jnp.int32),
        pltpu.VMEM((2, K, D), jnp.float32),
        pltpu.VMEM((2, K, D), jnp.float32),
        pltpu.VMEM((2, K, D), jnp.float32),
        pltpu.SemaphoreType.DMA,
        pltpu.SemaphoreType.DMA,
        pltpu.SemaphoreType.DMA,
        pltpu.SemaphoreType.DMA,
    ],
    compiler_params=_cp,
)
def _decode(h_hbm, src_hbm, dst_hbm, q_hbm,
            src_v, dst_v, a_v, b_v, q_v, g0, g1, o0, o1):
    wid = _worker_id()
    gs = (g0, g1)
    os = (o0, o1)

    pltpu.sync_copy(src_hbm.at[wid], src_v)
    pltpu.sync_copy(dst_hbm.at[wid], dst_v)

    def start_gather(j, p):
        pltpu.async_copy(h_hbm.at[src_v.at[j]], a_v.at[p], gs[p])
        pltpu.async_copy(h_hbm.at[dst_v.at[j]], b_v.at[p], gs[p])

    def wait_gather(j, p):
        pltpu.make_async_copy(h_hbm.at[src_v.at[j]], a_v.at[p], gs[p]).wait()
        pltpu.make_async_copy(h_hbm.at[dst_v.at[j]], b_v.at[p], gs[p]).wait()

    def start_out(j, p):
        pltpu.async_copy(q_v.at[p], q_hbm.at[wid, j], os[p])

    def wait_out(j, p):
        pltpu.make_async_copy(q_v.at[p], q_hbm.at[wid, j], os[p]).wait()

    def compute(j, p):
        @pl.loop(0, K, step=2)
        def _(e):
            for u in range(2):
                for t in range(D // 16):
                    sl = pl.ds(t * 16, 16)
                    q_v[p, e + u, sl] = a_v[p, e + u, sl] * b_v[p, e + u, sl]

    start_gather(0, 0)
    start_gather(1, 1)

    @pl.loop(0, NCH - 1, step=2)
    def _(i):
        wait_gather(i, 0)

        @pl.when(i >= 2)
        def _():
            wait_out(i - 2, 0)

        compute(i, 0)
        start_out(i, 0)
        start_gather(i + 2, 0)
        wait_gather(i + 1, 1)

        @pl.when(i >= 2)
        def _():
            wait_out(i - 1, 1)

        compute(i + 1, 1)
        start_out(i + 1, 1)

        @pl.when(i + 3 < NCH)
        def _():
            start_gather(i + 3, 1)

    wait_gather(NCH - 1, 0)
    wait_out(NCH - 3, 0)
    compute(NCH - 1, 0)
    start_out(NCH - 1, 0)
    wait_out(NCH - 2, 1)
    wait_out(NCH - 1, 0)


BE = 2000
EB = E // BE


def _finish_body(q_ref, et_ref, r_ref, out_ref):
    et = et_ref[0, 0]
    oh = (et[:, None] == lax.broadcasted_iota(jnp.int32, (BE, RP), 1))
    re = jnp.dot(oh.astype(jnp.float32), r_ref[...],
                 preferred_element_type=jnp.float32)
    out_ref[...] = jnp.sum(q_ref[...] * re, axis=1)[None, None, :]


_finish = pl.pallas_call(
    _finish_body,
    grid=(EB,),
    in_specs=[
        pl.BlockSpec((BE, D), lambda i: (i, 0)),
        pl.BlockSpec((1, 1, BE), lambda i: (i, 0, 0)),
        pl.BlockSpec((RP, D), lambda i: (0, 0)),
    ],
    out_specs=pl.BlockSpec((1, 1, BE), lambda i: (i, 0, 0)),
    out_shape=jax.ShapeDtypeStruct((EB, 1, BE), jnp.float32),
)


@jax.jit
def kernel(x, node_emb, rel_emb, W, W_loop, W_rel, edge_index, edge_type):
    del x
    src = edge_index[0]
    dst = edge_index[1]
    et = edge_type
    emb2 = node_emb.reshape(N * NC, DH)
    relh = jnp.pad(rel_emb, ((0, RP - R), (0, 0))).reshape(NC, RH, D)
    aggp = _encode(src.reshape(NS, NCHE, K), dst.reshape(NS, NCHE, K), emb2)
    cntp = _counts(dst.reshape(NS, NCHE, K), et.reshape(NS, NCHE, K))
    h, r = _dense(aggp, cntp, node_emb, relh, W, W_loop, W_rel)
    q = _decode(h, src.reshape(NW, NCH, K), dst.reshape(NW, NCH, K))
    scores = _finish(q.reshape(E, D), et.reshape(EB, 1, BE), r)
    return scores.reshape(E)
